# baseline (speedup 1.0000x reference)
import functools

import jax
import jax.numpy as jnp
from jax import lax
from jax.experimental import pallas as pl
from jax.experimental.pallas import tpu as pltpu
from jax.experimental.pallas import tpu_sc as plsc

_L = 16
_NW = 32
_ROWS = 32 * 12288
_ROWS_PER_W = _ROWS // _NW
_CHUNK = 2048
_NCHUNK = _ROWS_PER_W // _CHUNK
_GROUPS = _CHUNK // _L


def _tile_body(oe_hbm, et_hbm, lab_hbm, out_hbm, oe_v, et_v, lab_v, res_v):
    wid = lax.axis_index("s") * 2 + lax.axis_index("c")
    row0 = wid * _ROWS_PER_W

    idx5 = lax.iota(jnp.int32, _L) * 5

    def chunk(ci, carry):
        acc, cnt = carry
        base = (row0 + ci * _CHUNK) * 5
        pltpu.sync_copy(oe_hbm.at[pl.ds(base, _CHUNK * 5)], oe_v)
        pltpu.sync_copy(et_hbm.at[pl.ds(base, _CHUNK * 5)], et_v)
        pltpu.sync_copy(lab_hbm.at[pl.ds(row0 + ci * _CHUNK, _CHUNK)], lab_v)

        def group(g, carry):
            acc, cnt = carry
            gb = g * (5 * _L)
            ix = idx5 + gb
            dxo = plsc.load_gather(oe_v, [ix])
            dyo = plsc.load_gather(oe_v, [ix + 1])
            dlo = plsc.load_gather(oe_v, [ix + 2])
            dso = plsc.load_gather(oe_v, [ix + 3])
            to = plsc.load_gather(oe_v, [ix + 4])
            dxt = plsc.load_gather(et_v, [ix])
            dyt = plsc.load_gather(et_v, [ix + 1])
            dlt = plsc.load_gather(et_v, [ix + 2])
            dst = plsc.load_gather(et_v, [ix + 3])
            tt = plsc.load_gather(et_v, [ix + 4])
            lab = lab_v[pl.ds(g * _L, _L)]

            r_o = 1.0 / (1.0 + to * to)
            r_t = 1.0 / (1.0 + tt * tt)
            ct = 1.0 + to * tt
            st = to - tt
            c2 = ct * ct * r_o * r_t
            s2 = st * st * r_o * r_t
            e_lo = jnp.exp(-2.0 * dlo)
            e_so = jnp.exp(-2.0 * dso)
            e_lt = jnp.exp(2.0 * dlt)
            e_st = jnp.exp(2.0 * dst)
            trace = c2 * (e_lt * e_lo + e_st * e_so) \
                  + s2 * (e_lt * e_so + e_st * e_lo)
            u = dxo - dxt
            v = dyo - dyt
            a = u + to * v
            b = v - to * u
            dist = 4.0 * (a * a * e_lo + b * b * e_so) * r_o
            det = 2.0 * (dlo - dlt) + 2.0 * (dso - dst)
            kld = (trace + dist + det - 2.0) * 0.5
            pos = lab == 1
            acc = acc + jnp.where(pos, kld, 0.0)
            cnt = cnt + jnp.where(pos, 1.0, 0.0)
            return acc, cnt

        return lax.fori_loop(0, _GROUPS, group, (acc, cnt))

    zero = jnp.zeros((_L,), jnp.float32)
    acc, cnt = lax.fori_loop(0, _NCHUNK, chunk, (zero, zero))
    res_v[0] = acc
    res_v[1] = cnt
    pltpu.sync_copy(res_v, out_hbm.at[wid])


@jax.jit
def _loss(oe_flat, et_flat, lab_flat):
    mesh = plsc.VectorSubcoreMesh(core_axis_name="c", subcore_axis_name="s")
    parts = pl.kernel(
        _tile_body,
        mesh=mesh,
        compiler_params=pltpu.CompilerParams(needs_layout_passes=False),
        out_type=jax.ShapeDtypeStruct((_NW, 2, _L), jnp.float32),
        scratch_types=[
            pltpu.VMEM((_CHUNK * 5,), jnp.float32),
            pltpu.VMEM((_CHUNK * 5,), jnp.float32),
            pltpu.VMEM((_CHUNK,), jnp.int32),
            pltpu.VMEM((2, _L), jnp.float32),
        ],
    )(oe_flat, et_flat, lab_flat)
    total = jnp.sum(parts[:, 0, :])
    npos = jnp.sum(parts[:, 1, :])
    return total / jnp.maximum(npos, 1.0)


def kernel(out_ellipse, labels, ellipse_targets, anchors):
    oe_flat = out_ellipse.reshape(-1)
    et_flat = ellipse_targets.reshape(-1)
    lab_flat = labels.reshape(-1)
    return _loss(oe_flat, et_flat, lab_flat)
